# baseline (speedup 1.0000x reference)
import functools

import jax
import jax.numpy as jnp
from jax import lax
from jax.experimental import pallas as pl
from jax.experimental.pallas import tpu as pltpu
from jax.experimental.pallas import tpu_sc as plsc

N = 10000
NPAD = 10240
E = 320000
D = 128
DH = 64
DOUT = 40
DOUT_PAD = 64
NP = 10240

NUM_CORES = 2
NUM_SUBCORES = 16
ROWS_PER_TILE = NPAD // NUM_SUBCORES
CHUNK = 128
CH_TILE = 160
E_PAD = CH_TILE * NUM_SUBCORES * CHUNK
NCHUNKS = E_PAD // CHUNK
NBUF = 4
CH_Q = 40
QUADS = CH_TILE // NBUF

_SC_MESH = plsc.VectorSubcoreMesh(core_axis_name="c", subcore_axis_name="s")


def _pre_body(x_ref, w_ref, b_ref, lo_ref, hi_ref):
    h = jnp.dot(x_ref[...], w_ref[...], preferred_element_type=jnp.float32)
    h = h + b_ref[...]
    lo_ref[...] = h[:, :DH]
    hi_ref[...] = h[:, DH:]


def _pre_matmul(x, w0, b0r):
    bn = 1024
    return pl.pallas_call(
        _pre_body,
        grid=(NPAD // bn,),
        in_specs=[
            pl.BlockSpec((bn, D), lambda i: (i, 0)),
            pl.BlockSpec((D, D), lambda i: (0, 0)),
            pl.BlockSpec((1, D), lambda i: (0, 0)),
        ],
        out_specs=[
            pl.BlockSpec((bn, DH), lambda i: (i, 0)),
            pl.BlockSpec((bn, DH), lambda i: (i, 0)),
        ],
        out_shape=[
            jax.ShapeDtypeStruct((NPAD, DH), jnp.float32),
            jax.ShapeDtypeStruct((NPAD, DH), jnp.float32),
        ],
    )(x, w0, b0r)


def _hop_body(h_lo, h_hi, sc2d, ni2d, zeros_hbm,
              out_lo, out_hi,
              sc_v, ni_v, r0, r1, r2, r3, tab, acc,
              g0, g1, g2, g3, s0, s1, s2, s3):
    c = lax.axis_index("c")
    s = lax.axis_index("s")
    row0 = s * ROWS_PER_TILE
    cbase = s * CH_TILE
    rows = (r0, r1, r2, r3)
    gsem = (g0, g1, g2, g3)
    ssem = (s0, s1, s2, s3)

    def run(h_hbm, o_hbm):
        pltpu.sync_copy(h_hbm.at[pl.ds(row0, ROWS_PER_TILE)],
                        tab.at[pl.ds(row0, ROWS_PER_TILE)])
        pltpu.sync_copy(zeros_hbm.at[pl.ds(row0, ROWS_PER_TILE)],
                        acc.at[pl.ds(row0, ROWS_PER_TILE)])
        plsc.subcore_barrier()

        def start_g(j, b):
            pltpu.async_copy(tab.at[sc_v.at[j]], rows[b], gsem[b])

        def wait_g(j, b):
            pltpu.make_async_copy(tab.at[sc_v.at[j]], rows[b], gsem[b]).wait()

        def start_s(j, b):
            pltpu.async_copy(rows[b], acc.at[ni_v.at[j]], ssem[b], add=True)

        def wait_s(j, b):
            pltpu.make_async_copy(rows[b], acc.at[ni_v.at[j]], ssem[b]).wait()

        for q4 in range(CH_TILE // CH_Q):
            pltpu.sync_copy(sc2d.at[pl.ds(cbase + q4 * CH_Q, CH_Q)], sc_v)
            pltpu.sync_copy(ni2d.at[pl.ds(cbase + q4 * CH_Q, CH_Q)], ni_v)

            start_g(0, 0)
            for j in range(NBUF):
                if j + 1 < NBUF:
                    start_g(j + 1, j + 1)
                else:
                    wait_s(0, 0)
                    start_g(NBUF, 0)
                wait_g(j, j)
                start_s(j, j)

            @pl.loop(1, CH_Q // NBUF)
            def _quad(q):
                for u in range(NBUF):
                    j = NBUF * q + u
                    b, b1 = u, (u + 1) % NBUF

                    @pl.when(j + 1 < CH_Q)
                    def _():
                        wait_s(j - (NBUF - 1), b1)
                        start_g(j + 1, b1)

                    wait_g(j, b)
                    start_s(j, b)

            for u in range(NBUF):
                wait_s(CH_Q - NBUF + u, u)

        plsc.subcore_barrier()
        pltpu.sync_copy(acc.at[pl.ds(row0, ROWS_PER_TILE)],
                        o_hbm.at[pl.ds(row0, ROWS_PER_TILE)])

    @pl.when(c == 0)
    def _():
        run(h_lo, out_lo)

    @pl.when(c == 1)
    def _():
        run(h_hi, out_hi)


_hop_call = pl.kernel(
    _hop_body,
    out_type=[
        jax.ShapeDtypeStruct((NPAD, DH), jnp.float32),
        jax.ShapeDtypeStruct((NPAD, DH), jnp.float32),
    ],
    mesh=_SC_MESH,
    scratch_types=[
        pltpu.VMEM((CH_Q, CHUNK), jnp.int32),
        pltpu.VMEM((CH_Q, CHUNK), jnp.int32),
        pltpu.VMEM((CHUNK, DH), jnp.float32),
        pltpu.VMEM((CHUNK, DH), jnp.float32),
        pltpu.VMEM((CHUNK, DH), jnp.float32),
        pltpu.VMEM((CHUNK, DH), jnp.float32),
        pltpu.VMEM_SHARED((NPAD, DH), jnp.float32),
        pltpu.VMEM_SHARED((NPAD, DH), jnp.float32),
    ] + [pltpu.SemaphoreType.DMA] * 8,
    compiler_params=pltpu.CompilerParams(use_tc_tiling_on_sc=False),
)


def _head_body(scale_ref, p0a, p0b, p1a, p1b, p2a, p2b,
               wa, wb, wc, wd, we, wf, b_ref, out_ref):
    scale = scale_ref[0]
    acc = jnp.dot(p0a[...], wa[...], preferred_element_type=jnp.float32)
    acc += jnp.dot(p0b[...], wb[...], preferred_element_type=jnp.float32)
    acc *= scale
    acc += jnp.dot(p1a[...], wc[...], preferred_element_type=jnp.float32)
    acc += jnp.dot(p1b[...], wd[...], preferred_element_type=jnp.float32)
    acc += jnp.dot(p2a[...], we[...], preferred_element_type=jnp.float32)
    acc += jnp.dot(p2b[...], wf[...], preferred_element_type=jnp.float32)
    out_ref[...] = acc + b_ref[...]


def _head_matmul(scale, parts, wparts, bhr):
    bn = 1024
    part_spec = pl.BlockSpec((bn, DH), lambda i: (i, 0))
    w_spec = pl.BlockSpec((DH, DOUT_PAD), lambda i: (0, 0))
    return pl.pallas_call(
        _head_body,
        grid=(NPAD // bn,),
        in_specs=[pl.BlockSpec(memory_space=pltpu.SMEM)]
                 + [part_spec] * 6 + [w_spec] * 6
                 + [pl.BlockSpec((1, DOUT_PAD), lambda i: (0, 0))],
        out_specs=pl.BlockSpec((bn, DOUT_PAD), lambda i: (i, 0)),
        out_shape=jax.ShapeDtypeStruct((NPAD, DOUT_PAD), jnp.float32),
    )(scale, *parts, *wparts, bhr)


def _take_body(feat_hbm, nli_hbm, out_hbm, idx, rows, sem):
    w = lax.axis_index("s") * NUM_CORES + lax.axis_index("c")
    per_w = NP // (NUM_CORES * NUM_SUBCORES)
    base = w * per_w
    pltpu.sync_copy(nli_hbm.at[pl.ds(base, per_w)], idx)
    for lo, sz in ((0, CHUNK), (CHUNK, CHUNK), (2 * CHUNK, per_w - 2 * CHUNK)):
        pltpu.async_copy(feat_hbm.at[idx.at[pl.ds(lo, sz)]],
                         rows.at[pl.ds(lo, sz)], sem).wait()
    pltpu.sync_copy(rows, out_hbm.at[pl.ds(base, per_w)])


_take_call = pl.kernel(
    _take_body,
    out_type=jax.ShapeDtypeStruct((NP, DOUT_PAD), jnp.float32),
    mesh=_SC_MESH,
    scratch_types=[
        pltpu.VMEM((NP // (NUM_CORES * NUM_SUBCORES),), jnp.int32),
        pltpu.VMEM((NP // (NUM_CORES * NUM_SUBCORES), DOUT_PAD), jnp.float32),
        pltpu.SemaphoreType.DMA,
    ],
    compiler_params=pltpu.CompilerParams(use_tc_tiling_on_sc=False),
)


def kernel(x, agg_scatter_0, agg_node_index_0, agg_scatter_1, agg_node_index_1,
           node_label_index, node_label, W0, b0, eps, W_head, b_head):
    zeros = jnp.zeros((NPAD, DH), jnp.float32)

    x_pad = jnp.pad(x, ((0, NPAD - N), (0, 0)))
    h0_lo, h0_hi = _pre_matmul(x_pad, W0, b0.reshape(1, D))
    pad_idx = (N + jnp.arange(E_PAD - E, dtype=jnp.int32) % (NPAD - N))

    def _chunked(idx):
        return jnp.concatenate([idx, pad_idx]).reshape(NCHUNKS, CHUNK)

    h1_lo, h1_hi = _hop_call(h0_lo, h0_hi, _chunked(agg_scatter_0),
                             _chunked(agg_node_index_0), zeros)
    h2_lo, h2_hi = _hop_call(h1_lo, h1_hi, _chunked(agg_scatter_1),
                             _chunked(agg_node_index_1), zeros)

    w_pad = jnp.zeros((3 * D, DOUT_PAD), jnp.float32).at[:, :DOUT].set(W_head)
    wparts = [w_pad[k * DH:(k + 1) * DH] for k in range(6)]
    b_pad = jnp.zeros((1, DOUT_PAD), jnp.float32).at[0, :DOUT].set(b_head)
    scale = (1.0 + eps).astype(jnp.float32)

    feat = _head_matmul(scale,
                        (h0_lo, h0_hi, h1_lo, h1_hi, h2_lo, h2_hi),
                        wparts, b_pad)

    nli_pad = jnp.concatenate(
        [node_label_index, jnp.zeros((NP - N,), jnp.int32)])
    pred_pad = _take_call(feat, nli_pad)
    return (pred_pad[:N, :DOUT], node_label)


# speedup vs baseline: 1.0010x; 1.0010x over previous
import jax
import jax.numpy as jnp
from jax import lax
from jax.experimental import pallas as pl
from jax.experimental.pallas import tpu as pltpu
from jax.experimental.pallas import tpu_sc as plsc

N = 10000
NPAD = 10240
E = 320000
D = 128
DOUT = 40
DOUT_PAD = 64
NP = 10240

NUM_CORES = 2
NUM_SUBCORES = 16
ROWS_PER_TILE = NPAD // NUM_SUBCORES
CHUNK = 64
CH_TILE = 160
E_PAD = CH_TILE * NUM_SUBCORES * NUM_CORES * CHUNK
NCHUNKS = E_PAD // CHUNK
CH_H = CH_TILE // 2
NBUF = 4

_SC_MESH = plsc.VectorSubcoreMesh(core_axis_name="c", subcore_axis_name="s")


def _pre_body(x_ref, w_ref, b_ref, o_ref):
    o_ref[...] = jnp.dot(x_ref[...], w_ref[...],
                         preferred_element_type=jnp.float32) + b_ref[...]


def _pre_matmul(x, w0, b0r):
    bn = 1024
    return pl.pallas_call(
        _pre_body,
        grid=(NPAD // bn,),
        in_specs=[
            pl.BlockSpec((bn, D), lambda i: (i, 0)),
            pl.BlockSpec((D, D), lambda i: (0, 0)),
            pl.BlockSpec((1, D), lambda i: (0, 0)),
        ],
        out_specs=pl.BlockSpec((bn, D), lambda i: (i, 0)),
        out_shape=jax.ShapeDtypeStruct((NPAD, D), jnp.float32),
    )(x, w0, b0r)


def _hop_body(h_hbm, sc2d, ni2d, zeros_hbm,
              p0_hbm, p1_hbm,
              sc_v, ni_v, r0, r1, r2, r3, acc,
              g0, g1, g2, g3, s0, s1, s2, s3):
    c = lax.axis_index("c")
    s = lax.axis_index("s")
    row0 = s * ROWS_PER_TILE
    cbase = (c * NUM_SUBCORES + s) * CH_TILE
    rows = (r0, r1, r2, r3)
    gsem = (g0, g1, g2, g3)
    ssem = (s0, s1, s2, s3)

    pltpu.sync_copy(zeros_hbm.at[pl.ds(row0, ROWS_PER_TILE)],
                    acc.at[pl.ds(row0, ROWS_PER_TILE)])
    plsc.subcore_barrier()

    def start_g(j, b):
        pltpu.async_copy(h_hbm.at[sc_v.at[j]], rows[b], gsem[b])

    def wait_g(j, b):
        pltpu.make_async_copy(h_hbm.at[sc_v.at[j]], rows[b], gsem[b]).wait()

    def start_s(j, b):
        pltpu.async_copy(rows[b], acc.at[ni_v.at[j]], ssem[b], add=True)

    def wait_s(j, b):
        pltpu.make_async_copy(rows[b], acc.at[ni_v.at[j]], ssem[b]).wait()

    for hf in range(CH_TILE // CH_H):
        pltpu.sync_copy(sc2d.at[pl.ds(cbase + hf * CH_H, CH_H)], sc_v)
        pltpu.sync_copy(ni2d.at[pl.ds(cbase + hf * CH_H, CH_H)], ni_v)

        start_g(0, 0)
        for j in range(NBUF):
            if j + 1 < NBUF:
                start_g(j + 1, j + 1)
            else:
                wait_s(0, 0)
                start_g(NBUF, 0)
            wait_g(j, j)
            start_s(j, j)

        @pl.loop(1, CH_H // NBUF)
        def _quad(q):
            for u in range(NBUF):
                j = NBUF * q + u
                b, b1 = u, (u + 1) % NBUF

                @pl.when(j + 1 < CH_H)
                def _():
                    wait_s(j - (NBUF - 1), b1)
                    start_g(j + 1, b1)

                wait_g(j, b)
                start_s(j, b)

        for u in range(NBUF):
            wait_s(CH_H - NBUF + u, u)

    plsc.subcore_barrier()

    @pl.when(c == 0)
    def _():
        pltpu.sync_copy(acc.at[pl.ds(row0, ROWS_PER_TILE)],
                        p0_hbm.at[pl.ds(row0, ROWS_PER_TILE)])

    @pl.when(c == 1)
    def _():
        pltpu.sync_copy(acc.at[pl.ds(row0, ROWS_PER_TILE)],
                        p1_hbm.at[pl.ds(row0, ROWS_PER_TILE)])


_hop_call = pl.kernel(
    _hop_body,
    out_type=[
        jax.ShapeDtypeStruct((NPAD, D), jnp.float32),
        jax.ShapeDtypeStruct((NPAD, D), jnp.float32),
    ],
    mesh=_SC_MESH,
    scratch_types=[
        pltpu.VMEM((CH_H, CHUNK), jnp.int32),
        pltpu.VMEM((CH_H, CHUNK), jnp.int32),
        pltpu.VMEM((CHUNK, D), jnp.float32),
        pltpu.VMEM((CHUNK, D), jnp.float32),
        pltpu.VMEM((CHUNK, D), jnp.float32),
        pltpu.VMEM((CHUNK, D), jnp.float32),
        pltpu.VMEM_SHARED((NPAD, D), jnp.float32),
    ] + [pltpu.SemaphoreType.DMA] * 8,
    compiler_params=pltpu.CompilerParams(use_tc_tiling_on_sc=False),
)


def _add_body(a_ref, b_ref, o_ref):
    o_ref[...] = a_ref[...] + b_ref[...]


def _merge(a, b):
    bn = 1024
    spec = pl.BlockSpec((bn, D), lambda i: (i, 0))
    return pl.pallas_call(
        _add_body,
        grid=(NPAD // bn,),
        in_specs=[spec, spec],
        out_specs=spec,
        out_shape=jax.ShapeDtypeStruct((NPAD, D), jnp.float32),
    )(a, b)


def _head_body(scale_ref, h0_ref, h1_ref, h2_ref, w0_ref, w1_ref, w2_ref,
               b_ref, out_ref):
    acc = jnp.dot(h0_ref[...], w0_ref[...], preferred_element_type=jnp.float32)
    acc *= scale_ref[0]
    acc += jnp.dot(h1_ref[...], w1_ref[...], preferred_element_type=jnp.float32)
    acc += jnp.dot(h2_ref[...], w2_ref[...], preferred_element_type=jnp.float32)
    out_ref[...] = acc + b_ref[...]


def _head_matmul(scale, parts, wparts, bhr):
    bn = 1024
    part_spec = pl.BlockSpec((bn, D), lambda i: (i, 0))
    w_spec = pl.BlockSpec((D, DOUT_PAD), lambda i: (0, 0))
    return pl.pallas_call(
        _head_body,
        grid=(NPAD // bn,),
        in_specs=[pl.BlockSpec(memory_space=pltpu.SMEM)]
                 + [part_spec] * 3 + [w_spec] * 3
                 + [pl.BlockSpec((1, DOUT_PAD), lambda i: (0, 0))],
        out_specs=pl.BlockSpec((bn, DOUT_PAD), lambda i: (i, 0)),
        out_shape=jax.ShapeDtypeStruct((NPAD, DOUT_PAD), jnp.float32),
    )(scale, *parts, *wparts, bhr)


def _take_body(feat_hbm, nli_hbm, out_hbm, idx, rows, sem):
    w = lax.axis_index("s") * NUM_CORES + lax.axis_index("c")
    per_w = NP // (NUM_CORES * NUM_SUBCORES)
    base = w * per_w
    pltpu.sync_copy(nli_hbm.at[pl.ds(base, per_w)], idx)
    for lo, sz in ((0, 128), (128, 128), (256, per_w - 256)):
        pltpu.async_copy(feat_hbm.at[idx.at[pl.ds(lo, sz)]],
                         rows.at[pl.ds(lo, sz)], sem).wait()
    pltpu.sync_copy(rows, out_hbm.at[pl.ds(base, per_w)])


_take_call = pl.kernel(
    _take_body,
    out_type=jax.ShapeDtypeStruct((NP, DOUT_PAD), jnp.float32),
    mesh=_SC_MESH,
    scratch_types=[
        pltpu.VMEM((NP // (NUM_CORES * NUM_SUBCORES),), jnp.int32),
        pltpu.VMEM((NP // (NUM_CORES * NUM_SUBCORES), DOUT_PAD), jnp.float32),
        pltpu.SemaphoreType.DMA,
    ],
    compiler_params=pltpu.CompilerParams(use_tc_tiling_on_sc=False),
)


def kernel(x, agg_scatter_0, agg_node_index_0, agg_scatter_1, agg_node_index_1,
           node_label_index, node_label, W0, b0, eps, W_head, b_head):
    zeros = jnp.zeros((NPAD, D), jnp.float32)

    x_pad = jnp.pad(x, ((0, NPAD - N), (0, 0)))
    h0 = _pre_matmul(x_pad, W0, b0.reshape(1, D))

    pad_idx = (N + jnp.arange(E_PAD - E, dtype=jnp.int32) % (NPAD - N))

    def _chunked(idx):
        return jnp.concatenate([idx, pad_idx]).reshape(NCHUNKS, CHUNK)

    p0, p1 = _hop_call(h0, _chunked(agg_scatter_0),
                       _chunked(agg_node_index_0), zeros)
    h1 = _merge(p0, p1)
    p0, p1 = _hop_call(h1, _chunked(agg_scatter_1),
                       _chunked(agg_node_index_1), zeros)
    h2 = _merge(p0, p1)

    w_pad = jnp.zeros((3 * D, DOUT_PAD), jnp.float32).at[:, :DOUT].set(W_head)
    wparts = [w_pad[k * D:(k + 1) * D] for k in range(3)]
    b_pad = jnp.zeros((1, DOUT_PAD), jnp.float32).at[0, :DOUT].set(b_head)
    scale = (1.0 + eps).astype(jnp.float32)

    feat = _head_matmul(scale, (h0, h1, h2), wparts, b_pad)

    nli_pad = jnp.concatenate(
        [node_label_index, jnp.zeros((NP - N,), jnp.int32)])
    pred_pad = _take_call(feat, nli_pad)
    return (pred_pad[:N, :DOUT], node_label)


# speedup vs baseline: 1.0360x; 1.0350x over previous
import functools

import jax
import jax.numpy as jnp
from jax import lax
from jax.experimental import pallas as pl
from jax.experimental.pallas import tpu as pltpu
from jax.experimental.pallas import tpu_sc as plsc

N = 10000
NPAD = 10240
E = 320000
D = 128
DH = 64
DOUT = 40
DOUT_PAD = 64
NP = 10240

NUM_CORES = 2
NUM_SUBCORES = 16
ROWS_PER_TILE = NPAD // NUM_SUBCORES
CHUNK = 128
CH_TILE = 160
E_PAD = CH_TILE * NUM_SUBCORES * CHUNK
NCHUNKS = E_PAD // CHUNK
NBUF = 5
QUADS = CH_TILE // NBUF

_SC_MESH = plsc.VectorSubcoreMesh(core_axis_name="c", subcore_axis_name="s")


def _pre_body(x_ref, w_ref, b_ref, lo_ref, hi_ref):
    h = jnp.dot(x_ref[...], w_ref[...], preferred_element_type=jnp.float32)
    h = h + b_ref[...]
    lo_ref[...] = h[:, :DH]
    hi_ref[...] = h[:, DH:]


def _pre_matmul(x, w0, b0r):
    bn = 1024
    return pl.pallas_call(
        _pre_body,
        grid=(NPAD // bn,),
        in_specs=[
            pl.BlockSpec((bn, D), lambda i: (i, 0)),
            pl.BlockSpec((D, D), lambda i: (0, 0)),
            pl.BlockSpec((1, D), lambda i: (0, 0)),
        ],
        out_specs=[
            pl.BlockSpec((bn, DH), lambda i: (i, 0)),
            pl.BlockSpec((bn, DH), lambda i: (i, 0)),
        ],
        out_shape=[
            jax.ShapeDtypeStruct((NPAD, DH), jnp.float32),
            jax.ShapeDtypeStruct((NPAD, DH), jnp.float32),
        ],
    )(x, w0, b0r)


def _hop_body(h_lo, h_hi, sc2d, ni2d, zeros_hbm,
              out_lo, out_hi,
              sc_v, ni_v, r0, r1, r2, r3, r4, acc,
              g0, g1, g2, g3, g4, s0, s1, s2, s3, s4):
    c = lax.axis_index("c")
    s = lax.axis_index("s")
    row0 = s * ROWS_PER_TILE
    cbase = s * CH_TILE
    rows = (r0, r1, r2, r3, r4)
    gsem = (g0, g1, g2, g3, g4)
    ssem = (s0, s1, s2, s3, s4)

    def run(h_hbm, o_hbm):
        pltpu.sync_copy(sc2d.at[pl.ds(cbase, CH_TILE)], sc_v)
        pltpu.sync_copy(ni2d.at[pl.ds(cbase, CH_TILE)], ni_v)

        pltpu.sync_copy(zeros_hbm.at[pl.ds(row0, ROWS_PER_TILE)],
                        acc.at[pl.ds(row0, ROWS_PER_TILE)])
        plsc.subcore_barrier()

        def start_g(j, b):
            pltpu.async_copy(h_hbm.at[sc_v.at[j]], rows[b], gsem[b])

        def wait_g(j, b):
            pltpu.make_async_copy(h_hbm.at[sc_v.at[j]], rows[b], gsem[b]).wait()

        def start_s(j, b):
            pltpu.async_copy(rows[b], acc.at[ni_v.at[j]], ssem[b], add=True)

        def wait_s(j, b):
            pltpu.make_async_copy(rows[b], acc.at[ni_v.at[j]], ssem[b]).wait()

        start_g(0, 0)
        for j in range(NBUF):
            if j + 1 < NBUF:
                start_g(j + 1, j + 1)
            else:
                wait_s(0, 0)
                start_g(NBUF, 0)
            wait_g(j, j)
            start_s(j, j)

        @pl.loop(1, QUADS)
        def _quad(q):
            for u in range(NBUF):
                j = NBUF * q + u
                b, b1 = u, (u + 1) % NBUF

                @pl.when(j + 1 < CH_TILE)
                def _():
                    wait_s(j - (NBUF - 1), b1)
                    start_g(j + 1, b1)

                wait_g(j, b)
                start_s(j, b)

        for u in range(NBUF):
            wait_s(CH_TILE - NBUF + u, u)

        plsc.subcore_barrier()
        pltpu.sync_copy(acc.at[pl.ds(row0, ROWS_PER_TILE)],
                        o_hbm.at[pl.ds(row0, ROWS_PER_TILE)])

    @pl.when(c == 0)
    def _():
        run(h_lo, out_lo)

    @pl.when(c == 1)
    def _():
        run(h_hi, out_hi)


_hop_call = pl.kernel(
    _hop_body,
    out_type=[
        jax.ShapeDtypeStruct((NPAD, DH), jnp.float32),
        jax.ShapeDtypeStruct((NPAD, DH), jnp.float32),
    ],
    mesh=_SC_MESH,
    scratch_types=[
        pltpu.VMEM((CH_TILE, CHUNK), jnp.int32),
        pltpu.VMEM((CH_TILE, CHUNK), jnp.int32),
        pltpu.VMEM((CHUNK, DH), jnp.float32),
        pltpu.VMEM((CHUNK, DH), jnp.float32),
        pltpu.VMEM((CHUNK, DH), jnp.float32),
        pltpu.VMEM((CHUNK, DH), jnp.float32),
        pltpu.VMEM((CHUNK, DH), jnp.float32),
        pltpu.VMEM_SHARED((NPAD, DH), jnp.float32),
    ] + [pltpu.SemaphoreType.DMA] * 10,
    compiler_params=pltpu.CompilerParams(use_tc_tiling_on_sc=False),
)


def _head_body(scale_ref, p0a, p0b, p1a, p1b, p2a, p2b,
               wa, wb, wc, wd, we, wf, b_ref, out_ref):
    scale = scale_ref[0]
    acc = jnp.dot(p0a[...], wa[...], preferred_element_type=jnp.float32)
    acc += jnp.dot(p0b[...], wb[...], preferred_element_type=jnp.float32)
    acc *= scale
    acc += jnp.dot(p1a[...], wc[...], preferred_element_type=jnp.float32)
    acc += jnp.dot(p1b[...], wd[...], preferred_element_type=jnp.float32)
    acc += jnp.dot(p2a[...], we[...], preferred_element_type=jnp.float32)
    acc += jnp.dot(p2b[...], wf[...], preferred_element_type=jnp.float32)
    out_ref[...] = acc + b_ref[...]


def _head_matmul(scale, parts, wparts, bhr):
    bn = 1024
    part_spec = pl.BlockSpec((bn, DH), lambda i: (i, 0))
    w_spec = pl.BlockSpec((DH, DOUT_PAD), lambda i: (0, 0))
    return pl.pallas_call(
        _head_body,
        grid=(NPAD // bn,),
        in_specs=[pl.BlockSpec(memory_space=pltpu.SMEM)]
                 + [part_spec] * 6 + [w_spec] * 6
                 + [pl.BlockSpec((1, DOUT_PAD), lambda i: (0, 0))],
        out_specs=pl.BlockSpec((bn, DOUT_PAD), lambda i: (i, 0)),
        out_shape=jax.ShapeDtypeStruct((NPAD, DOUT_PAD), jnp.float32),
    )(scale, *parts, *wparts, bhr)


def _take_body(feat_hbm, nli_hbm, out_hbm, idx, rows, sem):
    w = lax.axis_index("s") * NUM_CORES + lax.axis_index("c")
    per_w = NP // (NUM_CORES * NUM_SUBCORES)
    base = w * per_w
    pltpu.sync_copy(nli_hbm.at[pl.ds(base, per_w)], idx)
    for lo, sz in ((0, CHUNK), (CHUNK, CHUNK), (2 * CHUNK, per_w - 2 * CHUNK)):
        pltpu.async_copy(feat_hbm.at[idx.at[pl.ds(lo, sz)]],
                         rows.at[pl.ds(lo, sz)], sem).wait()
    pltpu.sync_copy(rows, out_hbm.at[pl.ds(base, per_w)])


_take_call = pl.kernel(
    _take_body,
    out_type=jax.ShapeDtypeStruct((NP, DOUT_PAD), jnp.float32),
    mesh=_SC_MESH,
    scratch_types=[
        pltpu.VMEM((NP // (NUM_CORES * NUM_SUBCORES),), jnp.int32),
        pltpu.VMEM((NP // (NUM_CORES * NUM_SUBCORES), DOUT_PAD), jnp.float32),
        pltpu.SemaphoreType.DMA,
    ],
    compiler_params=pltpu.CompilerParams(use_tc_tiling_on_sc=False),
)


def kernel(x, agg_scatter_0, agg_node_index_0, agg_scatter_1, agg_node_index_1,
           node_label_index, node_label, W0, b0, eps, W_head, b_head):
    zeros = jnp.zeros((NPAD, DH), jnp.float32)

    x_pad = jnp.pad(x, ((0, NPAD - N), (0, 0)))
    h0_lo, h0_hi = _pre_matmul(x_pad, W0, b0.reshape(1, D))
    pad_idx = (N + jnp.arange(E_PAD - E, dtype=jnp.int32) % (NPAD - N))

    def _chunked(idx):
        return jnp.concatenate([idx, pad_idx]).reshape(NCHUNKS, CHUNK)

    h1_lo, h1_hi = _hop_call(h0_lo, h0_hi, _chunked(agg_scatter_0),
                             _chunked(agg_node_index_0), zeros)
    h2_lo, h2_hi = _hop_call(h1_lo, h1_hi, _chunked(agg_scatter_1),
                             _chunked(agg_node_index_1), zeros)

    w_pad = jnp.zeros((3 * D, DOUT_PAD), jnp.float32).at[:, :DOUT].set(W_head)
    wparts = [w_pad[k * DH:(k + 1) * DH] for k in range(6)]
    b_pad = jnp.zeros((1, DOUT_PAD), jnp.float32).at[0, :DOUT].set(b_head)
    scale = (1.0 + eps).astype(jnp.float32)

    feat = _head_matmul(scale,
                        (h0_lo, h0_hi, h1_lo, h1_hi, h2_lo, h2_hi),
                        wparts, b_pad)

    nli_pad = jnp.concatenate(
        [node_label_index, jnp.zeros((NP - N,), jnp.int32)])
    pred_pad = _take_call(feat, nli_pad)
    return (pred_pad[:N, :DOUT], node_label)


# speedup vs baseline: 1.0417x; 1.0055x over previous
import functools

import jax
import jax.numpy as jnp
from jax import lax
from jax.experimental import pallas as pl
from jax.experimental.pallas import tpu as pltpu
from jax.experimental.pallas import tpu_sc as plsc

N = 10000
NPAD = 10240
E = 320000
D = 128
DH = 64
DOUT = 40
DOUT_PAD = 128
NP = 10240

NUM_CORES = 2
NUM_SUBCORES = 16
ROWS_PER_TILE = NPAD // NUM_SUBCORES
CHUNK = 128
CH_TILE = 160
E_PAD = CH_TILE * NUM_SUBCORES * CHUNK
NCHUNKS = E_PAD // CHUNK
NBUF = 5
QUADS = CH_TILE // NBUF

_SC_MESH = plsc.VectorSubcoreMesh(core_axis_name="c", subcore_axis_name="s")


def _pre_body(x_ref, w_ref, b_ref, lo_ref, hi_ref):
    h = jnp.dot(x_ref[...], w_ref[...], preferred_element_type=jnp.float32)
    h = h + b_ref[...]
    lo_ref[...] = h[:, :DH]
    hi_ref[...] = h[:, DH:]


def _pre_matmul(x, w0, b0r):
    bn = 1024
    return pl.pallas_call(
        _pre_body,
        grid=(NPAD // bn,),
        in_specs=[
            pl.BlockSpec((bn, D), lambda i: (i, 0)),
            pl.BlockSpec((D, D), lambda i: (0, 0)),
            pl.BlockSpec((1, D), lambda i: (0, 0)),
        ],
        out_specs=[
            pl.BlockSpec((bn, DH), lambda i: (i, 0)),
            pl.BlockSpec((bn, DH), lambda i: (i, 0)),
        ],
        out_shape=[
            jax.ShapeDtypeStruct((NPAD, DH), jnp.float32),
            jax.ShapeDtypeStruct((NPAD, DH), jnp.float32),
        ],
    )(x, w0, b0r)


def _hop_body(h_lo, h_hi, sc2d, ni2d, zeros_hbm,
              out_lo, out_hi,
              sc_v, ni_v, r0, r1, r2, r3, r4, acc,
              g0, g1, g2, g3, g4, s0, s1, s2, s3, s4):
    c = lax.axis_index("c")
    s = lax.axis_index("s")
    row0 = s * ROWS_PER_TILE
    cbase = s * CH_TILE
    rows = (r0, r1, r2, r3, r4)
    gsem = (g0, g1, g2, g3, g4)
    ssem = (s0, s1, s2, s3, s4)

    def run(h_hbm, o_hbm):
        pltpu.sync_copy(sc2d.at[pl.ds(cbase, CH_TILE)], sc_v)
        pltpu.sync_copy(ni2d.at[pl.ds(cbase, CH_TILE)], ni_v)

        pltpu.sync_copy(zeros_hbm.at[pl.ds(row0, ROWS_PER_TILE)],
                        acc.at[pl.ds(row0, ROWS_PER_TILE)])
        plsc.subcore_barrier()

        def start_g(j, b):
            pltpu.async_copy(h_hbm.at[sc_v.at[j]], rows[b], gsem[b])

        def wait_g(j, b):
            pltpu.make_async_copy(h_hbm.at[sc_v.at[j]], rows[b], gsem[b]).wait()

        def start_s(j, b):
            pltpu.async_copy(rows[b], acc.at[ni_v.at[j]], ssem[b], add=True)

        def wait_s(j, b):
            pltpu.make_async_copy(rows[b], acc.at[ni_v.at[j]], ssem[b]).wait()

        start_g(0, 0)
        for j in range(NBUF):
            if j + 1 < NBUF:
                start_g(j + 1, j + 1)
            else:
                wait_s(0, 0)
                start_g(NBUF, 0)
            wait_g(j, j)
            start_s(j, j)

        @pl.loop(1, QUADS)
        def _quad(q):
            for u in range(NBUF):
                j = NBUF * q + u
                b, b1 = u, (u + 1) % NBUF

                @pl.when(j + 1 < CH_TILE)
                def _():
                    wait_s(j - (NBUF - 1), b1)
                    start_g(j + 1, b1)

                wait_g(j, b)
                start_s(j, b)

        for u in range(NBUF):
            wait_s(CH_TILE - NBUF + u, u)

        plsc.subcore_barrier()
        pltpu.sync_copy(acc.at[pl.ds(row0, ROWS_PER_TILE)],
                        o_hbm.at[pl.ds(row0, ROWS_PER_TILE)])

    @pl.when(c == 0)
    def _():
        run(h_lo, out_lo)

    @pl.when(c == 1)
    def _():
        run(h_hi, out_hi)


_hop_call = pl.kernel(
    _hop_body,
    out_type=[
        jax.ShapeDtypeStruct((NPAD, DH), jnp.float32),
        jax.ShapeDtypeStruct((NPAD, DH), jnp.float32),
    ],
    mesh=_SC_MESH,
    scratch_types=[
        pltpu.VMEM((CH_TILE, CHUNK), jnp.int32),
        pltpu.VMEM((CH_TILE, CHUNK), jnp.int32),
        pltpu.VMEM((CHUNK, DH), jnp.float32),
        pltpu.VMEM((CHUNK, DH), jnp.float32),
        pltpu.VMEM((CHUNK, DH), jnp.float32),
        pltpu.VMEM((CHUNK, DH), jnp.float32),
        pltpu.VMEM((CHUNK, DH), jnp.float32),
        pltpu.VMEM_SHARED((NPAD, DH), jnp.float32),
    ] + [pltpu.SemaphoreType.DMA] * 10,
    compiler_params=pltpu.CompilerParams(use_tc_tiling_on_sc=False),
)


def _head_body(scale_ref, p0a, p0b, p1a, p1b, p2a, p2b,
               wa, wb, wc, wd, we, wf, b_ref, out_ref):
    scale = scale_ref[0]
    acc = jnp.dot(p0a[...], wa[...], preferred_element_type=jnp.float32)
    acc += jnp.dot(p0b[...], wb[...], preferred_element_type=jnp.float32)
    acc *= scale
    acc += jnp.dot(p1a[...], wc[...], preferred_element_type=jnp.float32)
    acc += jnp.dot(p1b[...], wd[...], preferred_element_type=jnp.float32)
    acc += jnp.dot(p2a[...], we[...], preferred_element_type=jnp.float32)
    acc += jnp.dot(p2b[...], wf[...], preferred_element_type=jnp.float32)
    out_ref[...] = acc + b_ref[...]


def _head_matmul(scale, parts, wparts, bhr):
    bn = 1024
    part_spec = pl.BlockSpec((bn, DH), lambda i: (i, 0))
    w_spec = pl.BlockSpec((DH, DOUT_PAD), lambda i: (0, 0))
    return pl.pallas_call(
        _head_body,
        grid=(NPAD // bn,),
        in_specs=[pl.BlockSpec(memory_space=pltpu.SMEM)]
                 + [part_spec] * 6 + [w_spec] * 6
                 + [pl.BlockSpec((1, DOUT_PAD), lambda i: (0, 0))],
        out_specs=pl.BlockSpec((bn, DOUT_PAD), lambda i: (i, 0)),
        out_shape=jax.ShapeDtypeStruct((NPAD, DOUT_PAD), jnp.float32),
    )(scale, *parts, *wparts, bhr)


def _take_body(feat_hbm, nli_hbm, out_hbm, idx, rows, sem):
    w = lax.axis_index("s") * NUM_CORES + lax.axis_index("c")
    per_w = NP // (NUM_CORES * NUM_SUBCORES)
    base = w * per_w
    pltpu.sync_copy(nli_hbm.at[pl.ds(base, per_w)], idx)
    for lo, sz in ((0, CHUNK), (CHUNK, CHUNK), (2 * CHUNK, per_w - 2 * CHUNK)):
        pltpu.async_copy(feat_hbm.at[idx.at[pl.ds(lo, sz)]],
                         rows.at[pl.ds(lo, sz)], sem).wait()
    pltpu.sync_copy(rows, out_hbm.at[pl.ds(base, per_w)])


_take_call = pl.kernel(
    _take_body,
    out_type=jax.ShapeDtypeStruct((NP, DOUT_PAD), jnp.float32),
    mesh=_SC_MESH,
    scratch_types=[
        pltpu.VMEM((NP // (NUM_CORES * NUM_SUBCORES),), jnp.int32),
        pltpu.VMEM((NP // (NUM_CORES * NUM_SUBCORES), DOUT_PAD), jnp.float32),
        pltpu.SemaphoreType.DMA,
    ],
    compiler_params=pltpu.CompilerParams(use_tc_tiling_on_sc=False),
)


def kernel(x, agg_scatter_0, agg_node_index_0, agg_scatter_1, agg_node_index_1,
           node_label_index, node_label, W0, b0, eps, W_head, b_head):
    zeros = jnp.zeros((NPAD, DH), jnp.float32)

    x_pad = jnp.pad(x, ((0, NPAD - N), (0, 0)))
    h0_lo, h0_hi = _pre_matmul(x_pad, W0, b0.reshape(1, D))
    pad_idx = (N + jnp.arange(E_PAD - E, dtype=jnp.int32) % (NPAD - N))

    def _chunked(idx):
        return jnp.concatenate([idx, pad_idx]).reshape(NCHUNKS, CHUNK)

    h1_lo, h1_hi = _hop_call(h0_lo, h0_hi, _chunked(agg_scatter_0),
                             _chunked(agg_node_index_0), zeros)
    h2_lo, h2_hi = _hop_call(h1_lo, h1_hi, _chunked(agg_scatter_1),
                             _chunked(agg_node_index_1), zeros)

    w_pad = jnp.zeros((3 * D, DOUT_PAD), jnp.float32).at[:, :DOUT].set(W_head)
    wparts = [w_pad[k * DH:(k + 1) * DH] for k in range(6)]
    b_pad = jnp.zeros((1, DOUT_PAD), jnp.float32).at[0, :DOUT].set(b_head)
    scale = (1.0 + eps).astype(jnp.float32)

    feat = _head_matmul(scale,
                        (h0_lo, h0_hi, h1_lo, h1_hi, h2_lo, h2_hi),
                        wparts, b_pad)

    nli_pad = jnp.concatenate(
        [node_label_index, jnp.zeros((NP - N,), jnp.int32)])
    pred_pad = _take_call(feat, nli_pad)
    return (pred_pad[:N, :DOUT], node_label)
